# Initial kernel scaffold; baseline (speedup 1.0000x reference)
import jax
import jax.numpy as jnp
from jax.experimental import pallas as pl


def kernel(byte_window, bank):
    raise NotImplementedError("write your pallas kernel here")


# speedup vs baseline: 3.8550x; 3.8550x over previous
import functools

import jax
import jax.numpy as jnp
from jax import lax
from jax.experimental import pallas as pl
from jax.experimental.pallas import tpu as pltpu
from jax.experimental.pallas import tpu_sc as plsc

_N_SLOTS = 100000
_D_STATE = 8
_NGRAM = 16
_N_BLADES = 8

_NC = 2
_NS = 16
_NW = _NC * _NS
_LANES = 16

_CHUNK = 512
_GROUPS = _CHUNK // _LANES
_IDX_PER_DMA = 128
_DMAS = _CHUNK * _N_BLADES // _IDX_PER_DMA


def _sc_hash_gather(bw_flat, bank_flat, n_tok):
    tok_per_w = n_tok // _NW
    n_chunks = tok_per_w // _CHUNK
    mesh = plsc.VectorSubcoreMesh(
        core_axis_name="c", subcore_axis_name="s",
        num_cores=_NC, num_subcores=_NS)

    @functools.partial(
        pl.kernel,
        compiler_params=pltpu.CompilerParams(
            needs_layout_passes=False, use_tc_tiling_on_sc=False),
        out_type=jax.ShapeDtypeStruct((n_tok * _N_BLADES, _D_STATE),
                                      jnp.float32),
        mesh=mesh,
        scratch_types=[
            pltpu.VMEM((tok_per_w * _NGRAM,), jnp.int32),
            pltpu.VMEM((_CHUNK * _N_BLADES,), jnp.int32),
            pltpu.VMEM((_CHUNK * _N_BLADES, _D_STATE), jnp.float32),
            pltpu.SemaphoreType.DMA,
        ],
    )
    def k(bw_hbm, bank_hbm, out_hbm, bw_v, idx_v, rows_v, sem):
        wid = lax.axis_index("s") * _NC + lax.axis_index("c")
        tok0 = wid * tok_per_w
        pltpu.sync_copy(
            bw_hbm.at[pl.ds(tok0 * _NGRAM, tok_per_w * _NGRAM)], bw_v)
        lane = lax.iota(jnp.int32, _LANES)

        for c in range(n_chunks):
            def group(g, carry):
                tloc = (c * _CHUNK + g * _LANES) + lane
                byte_base = tloc * _NGRAM
                h = jnp.full((_LANES,), 2166136261, dtype=jnp.uint32)
                for i in range(_NGRAM):
                    b = plsc.load_gather(bw_v, [byte_base + i])
                    h = (h ^ b.astype(jnp.uint32)) * jnp.uint32(16777619)
                addr = (h % jnp.uint32(_N_SLOTS)).astype(jnp.int32)
                pos = (g * _LANES + lane) * _N_BLADES
                for blade in range(_N_BLADES):
                    plsc.store_scatter(idx_v, [pos + blade],
                                       addr + blade * _N_SLOTS)
                return carry
            lax.fori_loop(0, _GROUPS, group, 0)

            copies = []
            for j in range(_DMAS):
                copies.append(pltpu.async_copy(
                    bank_hbm.at[idx_v.at[pl.ds(j * _IDX_PER_DMA,
                                               _IDX_PER_DMA)]],
                    rows_v.at[pl.ds(j * _IDX_PER_DMA, _IDX_PER_DMA)],
                    sem))
            for cp in copies:
                cp.wait()
            pltpu.sync_copy(
                rows_v,
                out_hbm.at[pl.ds((tok0 + c * _CHUNK) * _N_BLADES,
                                 _CHUNK * _N_BLADES)])

    return k(bw_flat, bank_flat)


def kernel(byte_window, bank):
    B, S, _ = byte_window.shape
    n_tok = B * S
    bw_flat = byte_window.reshape(-1)
    bank_flat = bank.reshape(_N_BLADES * _N_SLOTS, _D_STATE)
    out = _sc_hash_gather(bw_flat, bank_flat, n_tok)
    return out.reshape(B, S, _N_BLADES, _D_STATE)


# speedup vs baseline: 19.8492x; 5.1490x over previous
import functools

import jax
import jax.numpy as jnp
from jax import lax
from jax.experimental import pallas as pl
from jax.experimental.pallas import tpu as pltpu
from jax.experimental.pallas import tpu_sc as plsc

_N_SLOTS = 100000
_D_STATE = 8
_NGRAM = 16
_N_BLADES = 8

_NC = 2
_NS = 16
_LANES = 16

_B = 16
_S = 4096
_BW_CHUNK = 512


def _sc_hash_gather(bw_t, bank_t):
    mesh = plsc.VectorSubcoreMesh(
        core_axis_name="c", subcore_axis_name="s",
        num_cores=_NC, num_subcores=_NS)

    @functools.partial(
        pl.kernel,
        compiler_params=pltpu.CompilerParams(
            needs_layout_passes=False, use_tc_tiling_on_sc=True),
        out_type=jax.ShapeDtypeStruct((_B, _N_BLADES, _D_STATE, _S),
                                      jnp.float32),
        mesh=mesh,
        scratch_types=[
            pltpu.VMEM((_N_SLOTS,), jnp.float32),
            pltpu.VMEM((_NGRAM, _BW_CHUNK), jnp.int32),
            pltpu.VMEM((_S,), jnp.int32),
            pltpu.VMEM((_S,), jnp.float32),
            pltpu.VMEM_SHARED((_B * _S,), jnp.int32),
        ],
    )
    def k(bw_hbm, bank_hbm, out_hbm, table_v, bw_v, addr_v, ov, addr_sh):
        cid = lax.axis_index("c")
        sid = lax.axis_index("s")
        lane = lax.iota(jnp.int32, _LANES)

        for c in range(_S // _BW_CHUNK):
            pltpu.sync_copy(
                bw_hbm.at[sid, :, pl.ds(c * _BW_CHUNK, _BW_CHUNK)], bw_v)

            def group(g, carry, c=c):
                h = jnp.full((_LANES,), 2166136261, dtype=jnp.uint32)
                for i in range(_NGRAM):
                    byte = bw_v[i, pl.ds(g * _LANES, _LANES)]
                    h = (h ^ byte.astype(jnp.uint32)) * jnp.uint32(16777619)
                addr = (h % jnp.uint32(_N_SLOTS)).astype(jnp.int32)
                addr_v[pl.ds(c * _BW_CHUNK + g * _LANES, _LANES)] = addr
                return carry
            lax.fori_loop(0, _BW_CHUNK // _LANES, group, 0)
        pltpu.sync_copy(addr_v, addr_sh.at[pl.ds(sid * _S, _S)])
        plsc.subcore_barrier()

        w = cid * _NS + sid
        for t in range(2):
            p = w * 2 + t
            blade = p // _D_STATE
            d = p % _D_STATE
            pltpu.sync_copy(bank_hbm.at[blade, d, :], table_v)
            for b in range(_B):
                pltpu.sync_copy(addr_sh.at[pl.ds(b * _S, _S)], addr_v)

                def gather(g, carry):
                    idx = addr_v[pl.ds(g * _LANES, _LANES)]
                    ov[pl.ds(g * _LANES, _LANES)] = (
                        plsc.load_gather(table_v, [idx]))
                    return carry
                lax.fori_loop(0, _S // _LANES, gather, 0)
                pltpu.sync_copy(ov, out_hbm.at[b, blade, d, :])

    return k(bw_t, bank_t)


def kernel(byte_window, bank):
    bw_t = jnp.transpose(byte_window, (0, 2, 1))
    bank_t = jnp.transpose(bank, (0, 2, 1))
    out_t = _sc_hash_gather(bw_t, bank_t)
    return jnp.transpose(out_t, (0, 3, 1, 2))
